# baseline (speedup 1.0000x reference)
import functools
import jax
import jax.numpy as jnp
from jax import lax
from jax.experimental import pallas as pl
from jax.experimental.pallas import tpu as pltpu
from jax.experimental.pallas import tpu_sc as plsc

_N = 10000
_E = 320000
_NC = 2
_NS = 16
_K = 100
_RSTRIDE = 624
_RSPAN = 640

_NCH_FS = _E // (_NS * _K)
_NCH_ES = _E // (_NC * _NS * _K)

_MESH = plsc.VectorSubcoreMesh(core_axis_name="c", subcore_axis_name="s")


def _make_seg64():
    R = 4

    @functools.partial(
        pl.kernel,
        out_type=jax.ShapeDtypeStruct((_NC, _N, 64), jnp.float32),
        mesh=_MESH,
        compiler_params=pltpu.CompilerParams(use_tc_tiling_on_sc=False),
        scratch_types=[
            pltpu.VMEM((_NCH_FS, _K), jnp.int32),
            pltpu.VMEM((_NCH_FS, _K), jnp.int32),
            [pltpu.VMEM((_K, 64), jnp.float32)] * R,
            pltpu.VMEM_SHARED((_N, 64), jnp.float32),
            [pltpu.SemaphoreType.DMA] * R,
            [pltpu.SemaphoreType.DMA] * R,
        ],
    )
    def seg(table_hbm, gsrc_hbm, gdst_hbm, zeros_hbm, out_hbm,
            src_v, dst_v, rows, acc_sh, gsem, ssem):
        c = lax.axis_index("c")
        s = lax.axis_index("s")
        pltpu.sync_copy(zeros_hbm, acc_sh.at[pl.ds(s * _RSTRIDE, _RSPAN)])
        pltpu.sync_copy(gsrc_hbm.at[c, s], src_v)
        pltpu.sync_copy(gdst_hbm.at[c, s], dst_v)
        plsc.subcore_barrier()

        for b in range(R):
            pltpu.async_copy(table_hbm.at[src_v.at[b]], rows[b], gsem[b])

        def body(j, _):
            for b in range(R):
                i = j * R + b
                pltpu.make_async_copy(table_hbm.at[src_v.at[i]],
                                      rows[b], gsem[b]).wait()
                pltpu.async_copy(rows[b], acc_sh.at[dst_v.at[i]],
                                 ssem[b], add=True)
            for b in range(R):
                i = j * R + b
                pltpu.make_async_copy(rows[b], acc_sh.at[dst_v.at[i]],
                                      ssem[b]).wait()
                nxt = i + R

                @pl.when(nxt < _NCH_FS)
                def _():
                    pltpu.async_copy(table_hbm.at[src_v.at[nxt]],
                                     rows[b], gsem[b])
            return _

        lax.fori_loop(0, _NCH_FS // R, body, None)
        plsc.subcore_barrier()
        pltpu.sync_copy(acc_sh.at[pl.ds(s * _RSTRIDE, _RSPAN)],
                        out_hbm.at[c, pl.ds(s * _RSTRIDE, _RSPAN)])

    return seg


def _make_seg64x2():
    R = 4

    @functools.partial(
        pl.kernel,
        out_type=jax.ShapeDtypeStruct((2, _NC, _N, 64), jnp.float32),
        mesh=_MESH,
        compiler_params=pltpu.CompilerParams(use_tc_tiling_on_sc=False),
        scratch_types=[
            pltpu.VMEM((_NCH_FS, _K), jnp.int32),
            pltpu.VMEM((_NCH_FS, _K), jnp.int32),
            pltpu.VMEM((_NCH_FS, _K), jnp.int32),
            [pltpu.VMEM((_K, 64), jnp.float32)] * R,
            pltpu.VMEM_SHARED((_N, 64), jnp.float32),
            [pltpu.SemaphoreType.DMA] * R,
            [pltpu.SemaphoreType.DMA] * R,
        ],
    )
    def seg(table_hbm, gA_hbm, gB_hbm, gdst_hbm, zeros_hbm, out_hbm,
            srcA_v, srcB_v, dst_v, rows, acc_sh, gsem, ssem):
        c = lax.axis_index("c")
        s = lax.axis_index("s")
        pltpu.sync_copy(zeros_hbm, acc_sh.at[pl.ds(s * _RSTRIDE, _RSPAN)])
        pltpu.sync_copy(gA_hbm.at[c, s], srcA_v)
        pltpu.sync_copy(gB_hbm.at[c, s], srcB_v)
        pltpu.sync_copy(gdst_hbm.at[c, s], dst_v)
        plsc.subcore_barrier()

        def run_pass(src_v, acc_sh):
            for b in range(R):
                pltpu.async_copy(table_hbm.at[src_v.at[b]], rows[b], gsem[b])

            def body(j, _):
                for b in range(R):
                    i = j * R + b
                    pltpu.make_async_copy(table_hbm.at[src_v.at[i]],
                                          rows[b], gsem[b]).wait()
                    pltpu.async_copy(rows[b], acc_sh.at[dst_v.at[i]],
                                     ssem[b], add=True)
                for b in range(R):
                    i = j * R + b
                    pltpu.make_async_copy(rows[b], acc_sh.at[dst_v.at[i]],
                                          ssem[b]).wait()
                    nxt = i + R

                    @pl.when(nxt < _NCH_FS)
                    def _():
                        pltpu.async_copy(table_hbm.at[src_v.at[nxt]],
                                         rows[b], gsem[b])
                return _

            lax.fori_loop(0, _NCH_FS // R, body, None)

        run_pass(srcA_v, acc_sh)
        plsc.subcore_barrier()
        pltpu.sync_copy(acc_sh.at[pl.ds(s * _RSTRIDE, _RSPAN)],
                        out_hbm.at[0, c, pl.ds(s * _RSTRIDE, _RSPAN)])
        plsc.subcore_barrier()
        pltpu.sync_copy(zeros_hbm, acc_sh.at[pl.ds(s * _RSTRIDE, _RSPAN)])
        plsc.subcore_barrier()
        run_pass(srcB_v, acc_sh)
        plsc.subcore_barrier()
        pltpu.sync_copy(acc_sh.at[pl.ds(s * _RSTRIDE, _RSPAN)],
                        out_hbm.at[1, c, pl.ds(s * _RSTRIDE, _RSPAN)])

    return seg


def _make_seg16():
    R = 5

    @functools.partial(
        pl.kernel,
        out_type=jax.ShapeDtypeStruct((_NC, _N, 16), jnp.float32),
        mesh=_MESH,
        compiler_params=pltpu.CompilerParams(use_tc_tiling_on_sc=False),
        scratch_types=[
            pltpu.VMEM((_NCH_ES, _K), jnp.int32),
            pltpu.VMEM((_NCH_ES, _K), jnp.int32),
            [pltpu.VMEM((_K, 16), jnp.float32)] * R,
            pltpu.VMEM_SHARED((_N, 16), jnp.float32),
            [pltpu.SemaphoreType.DMA] * R,
            [pltpu.SemaphoreType.DMA] * R,
        ],
    )
    def seg(table_hbm, gsrc_hbm, gdst_hbm, zeros_hbm, out_hbm,
            src_v, dst_v, rows, acc_sh, gsem, ssem):
        c = lax.axis_index("c")
        s = lax.axis_index("s")
        pltpu.sync_copy(zeros_hbm, acc_sh.at[pl.ds(s * _RSTRIDE, _RSPAN)])
        pltpu.sync_copy(gsrc_hbm.at[c, s], src_v)
        pltpu.sync_copy(gdst_hbm.at[c, s], dst_v)
        plsc.subcore_barrier()

        for b in range(R):
            pltpu.async_copy(table_hbm.at[src_v.at[b]], rows[b], gsem[b])

        def body(j, _):
            for b in range(R):
                i = j * R + b
                pltpu.make_async_copy(table_hbm.at[src_v.at[i]],
                                      rows[b], gsem[b]).wait()
                pltpu.async_copy(rows[b], acc_sh.at[dst_v.at[i]],
                                 ssem[b], add=True)
            for b in range(R):
                i = j * R + b
                pltpu.make_async_copy(rows[b], acc_sh.at[dst_v.at[i]],
                                      ssem[b]).wait()
                nxt = i + R

                @pl.when(nxt < _NCH_ES)
                def _():
                    pltpu.async_copy(table_hbm.at[src_v.at[nxt]],
                                     rows[b], gsem[b])
            return _

        lax.fori_loop(0, _NCH_ES // R, body, None)
        plsc.subcore_barrier()
        pltpu.sync_copy(acc_sh.at[pl.ds(s * _RSTRIDE, _RSPAN)],
                        out_hbm.at[c, pl.ds(s * _RSTRIDE, _RSPAN)])

    return seg


def _make_deg():
    @functools.partial(
        pl.kernel,
        out_type=(jax.ShapeDtypeStruct((_NC, _N, 16), jnp.float32),
                  jax.ShapeDtypeStruct((_NC, _N, 16), jnp.float32)),
        mesh=_MESH,
        compiler_params=pltpu.CompilerParams(use_tc_tiling_on_sc=False),
        scratch_types=[
            pltpu.VMEM((_NCH_ES, _K), jnp.int32),
            pltpu.VMEM((_NCH_ES, _K), jnp.int32),
            pltpu.VMEM((_K, 16), jnp.float32),
            pltpu.VMEM_SHARED((_N, 16), jnp.float32),
            pltpu.VMEM_SHARED((_N, 16), jnp.float32),
            [pltpu.SemaphoreType.DMA] * 5,
            [pltpu.SemaphoreType.DMA] * 5,
        ],
    )
    def deg(gsrc_hbm, gdst_hbm, ones_hbm, zeros_hbm, od_hbm, id_hbm,
            src_v, dst_v, rows_v, accs_sh, accd_sh, ssem, dsem):
        R = 5
        c = lax.axis_index("c")
        s = lax.axis_index("s")
        pltpu.sync_copy(zeros_hbm, accs_sh.at[pl.ds(s * _RSTRIDE, _RSPAN)])
        pltpu.sync_copy(zeros_hbm, accd_sh.at[pl.ds(s * _RSTRIDE, _RSPAN)])
        pltpu.sync_copy(ones_hbm, rows_v)
        pltpu.sync_copy(gsrc_hbm.at[c, s], src_v)
        pltpu.sync_copy(gdst_hbm.at[c, s], dst_v)
        plsc.subcore_barrier()

        for b in range(R):
            pltpu.async_copy(rows_v, accs_sh.at[src_v.at[b]], ssem[b],
                             add=True)
            pltpu.async_copy(rows_v, accd_sh.at[dst_v.at[b]], dsem[b],
                             add=True)

        def body(j, _):
            for b in range(R):
                i = j * R + b
                pltpu.make_async_copy(rows_v, accs_sh.at[src_v.at[i]],
                                      ssem[b]).wait()
                pltpu.make_async_copy(rows_v, accd_sh.at[dst_v.at[i]],
                                      dsem[b]).wait()
                nxt = i + R

                @pl.when(nxt < _NCH_ES)
                def _():
                    pltpu.async_copy(rows_v, accs_sh.at[src_v.at[nxt]],
                                     ssem[b], add=True)
                    pltpu.async_copy(rows_v, accd_sh.at[dst_v.at[nxt]],
                                     dsem[b], add=True)
            return _

        lax.fori_loop(0, _NCH_ES // 5, body, None)
        plsc.subcore_barrier()
        pltpu.sync_copy(accs_sh.at[pl.ds(s * _RSTRIDE, _RSPAN)],
                        od_hbm.at[c, pl.ds(s * _RSTRIDE, _RSPAN)])
        pltpu.sync_copy(accd_sh.at[pl.ds(s * _RSTRIDE, _RSPAN)],
                        id_hbm.at[c, pl.ds(s * _RSTRIDE, _RSPAN)])

    return deg


_BM = 1000
_G = _N // _BM


def _rb(d):
    return pl.BlockSpec((_BM, d), lambda i: (i, 0))


def _full(shape):
    nd = len(shape)
    return pl.BlockSpec(shape, lambda i: (0,) * nd)


def _prep_scale(od, idg, in_feat):
    def body(od0, od1, id0, id1, x_ref, ro_ref, ri_ref, t1_ref):
        odeg = jnp.maximum(od0[...] + od1[...], 1.0)
        ideg = jnp.maximum(id0[...] + id1[...], 1.0)
        ro = lax.rsqrt(odeg)
        ro_ref[...] = ro
        ri_ref[...] = lax.rsqrt(ideg)
        t1_ref[...] = x_ref[...] * ro[:, :1]

    return pl.pallas_call(
        body,
        grid=(_G,),
        in_specs=[_rb(16)] * 4 + [_rb(128)],
        out_specs=[_rb(16), _rb(16), _rb(128)],
        out_shape=[jax.ShapeDtypeStruct((_N, 16), jnp.float32)] * 2
        + [jax.ShapeDtypeStruct((_N, 128), jnp.float32)],
    )(od[0], od[1], idg[0], idg[1], in_feat)


def _stage_mm(agg, ri, ro, W, b):
    do = W.shape[1]

    def body(a0, a1, ri_ref, ro_ref, w_ref, b_ref, o_ref):
        x0 = a0[...] * ri_ref[:, :1]
        x1 = a1[...] * ri_ref[:, :1]
        y = (jnp.dot(x0, w_ref[0:64, :], preferred_element_type=jnp.float32)
             + jnp.dot(x1, w_ref[64:128, :],
                       preferred_element_type=jnp.float32))
        o_ref[...] = jax.nn.relu(y + b_ref[...]) * ro_ref[:, :1]

    return pl.pallas_call(
        body,
        grid=(_G,),
        in_specs=[_rb(64), _rb(64), _rb(16), _rb(16),
                  _full((128, do)), _full((1, do))],
        out_specs=_rb(do),
        out_shape=jax.ShapeDtypeStruct((_N, do), jnp.float32),
    )(agg[0], agg[1], ri, ro, W, b[None, :])


def _stage_l3l4(aggA, aggB, ri, ro, W3, b3, W4):
    def body(a0, a1, a2, a3, ri_ref, ro_ref, w3_ref, b3_ref, w4_ref, o_ref):
        t = None
        for q, aq in enumerate((a0, a1, a2, a3)):
            xq = aq[...] * ri_ref[:, :1]
            yq = jnp.dot(xq, w3_ref[64 * q:64 * (q + 1), :],
                         preferred_element_type=jnp.float32)
            t = yq if t is None else t + yq
        t = jax.nn.relu(t + b3_ref[...]) * ro_ref[:, :1]
        o_ref[...] = jnp.dot(t, w4_ref[...],
                             preferred_element_type=jnp.float32)

    return pl.pallas_call(
        body,
        grid=(_G,),
        in_specs=[_rb(64)] * 4 + [_rb(16), _rb(16),
                                  _full((256, 256)), _full((1, 256)),
                                  _full((256, 128))],
        out_specs=_rb(128),
        out_shape=jax.ShapeDtypeStruct((_N, 128), jnp.float32),
    )(aggA[0], aggA[1], aggB[0], aggB[1], ri, ro, W3, b3[None, :], W4)


def _stage_l4l5(agg, ri, ro, b4, W5):
    def body(a0, a1, ri_ref, ro_ref, b4_ref, w50_ref, w51_ref, o_ref):
        h0 = jax.nn.relu(a0[...] * ri_ref[:, :1] + b4_ref[:, 0:64])
        h1 = jax.nn.relu(a1[...] * ri_ref[:, :1] + b4_ref[:, 64:128])
        h0 = h0 * ro_ref[:, :1]
        h1 = h1 * ro_ref[:, :1]
        o_ref[...] = (
            jnp.dot(h0, w50_ref[...], preferred_element_type=jnp.float32)
            + jnp.dot(h1, w51_ref[...], preferred_element_type=jnp.float32))

    W5b = jnp.broadcast_to(W5, (128, 16))
    return pl.pallas_call(
        body,
        grid=(_G,),
        in_specs=[_rb(64), _rb(64), _rb(16), _rb(16),
                  _full((1, 128)), _full((64, 16)), _full((64, 16))],
        out_specs=_rb(16),
        out_shape=jax.ShapeDtypeStruct((_N, 16), jnp.float32),
    )(agg[0], agg[1], ri, ro, b4[None, :], W5b[0:64], W5b[64:128])


def _stage_out(agg, ri, b5):
    def body(a0, a1, ri_ref, b5_ref, o_ref):
        v = (a0[...] + a1[...]) * ri_ref[...]
        o_ref[...] = jax.nn.relu(v + b5_ref[0, 0])

    return pl.pallas_call(
        body,
        grid=(_G,),
        in_specs=[_rb(16), _rb(16), _rb(16), _full((1, 1))],
        out_specs=_rb(16),
        out_shape=jax.ShapeDtypeStruct((_N, 16), jnp.float32),
    )(agg[0], agg[1], ri, b5[None, :])


def kernel(in_feat, edge_index, W1, b1, W2, b2, W3, b3, W4, b4, W5, b5):
    ei = edge_index.astype(jnp.int32)
    src, dst = ei[0], ei[1]

    def fs(a, b):
        return jnp.stack([a, b]).reshape(_NC, _NS, _NCH_FS, _K)

    g_h = fs(2 * src, 2 * src + 1)
    gA = fs(4 * src, 4 * src + 1)
    gB = fs(4 * src + 2, 4 * src + 3)
    gd_fs = fs(dst, dst)
    src_es = src.reshape(_NC, _NS, _NCH_ES, _K)
    dst_es = dst.reshape(_NC, _NS, _NCH_ES, _K)

    z64 = jnp.zeros((_RSPAN, 64), jnp.float32)
    z16 = jnp.zeros((_RSPAN, 16), jnp.float32)
    ones16 = jnp.ones((_K, 16), jnp.float32)

    seg64 = _make_seg64()
    seg64x2 = _make_seg64x2()
    seg16 = _make_seg16()
    deg = _make_deg()

    od, idg = deg(src_es, dst_es, ones16, z16)
    ro, ri, t1 = _prep_scale(od, idg, in_feat)
    a1 = seg64(t1.reshape(_NC * _N, 64), g_h, gd_fs, z64)
    t2 = _stage_mm(a1, ri, ro, W1, b1)
    a2 = seg64(t2.reshape(_NC * _N, 64), g_h, gd_fs, z64)
    t3 = _stage_mm(a2, ri, ro, W2, b2)
    t3q = t3.reshape(4 * _N, 64)
    a3 = seg64x2(t3q, gA, gB, gd_fs, z64)
    u = _stage_l3l4(a3[0], a3[1], ri, ro, W3, b3, W4)
    a4 = seg64(u.reshape(_NC * _N, 64), g_h, gd_fs, z64)
    v16 = _stage_l4l5(a4, ri, ro, b4, W5)
    a5 = seg16(v16, src_es, dst_es, z16)
    out16 = _stage_out(a5, ri, b5)
    return out16[:, :1]


# speedup vs baseline: 1.0298x; 1.0298x over previous
import functools
import jax
import jax.numpy as jnp
from jax import lax
from jax.experimental import pallas as pl
from jax.experimental.pallas import tpu as pltpu
from jax.experimental.pallas import tpu_sc as plsc

_N = 10000
_E = 320000
_NC = 2
_NS = 16
_K = 80
_RSTRIDE = 624
_RSPAN = 640

_NCH_FS = _E // (_NS * _K)
_NCH_ES = _E // (_NC * _NS * _K)

_MESH = plsc.VectorSubcoreMesh(core_axis_name="c", subcore_axis_name="s")


def _make_seg64():
    R = 5

    @functools.partial(
        pl.kernel,
        out_type=jax.ShapeDtypeStruct((_NC, _N, 64), jnp.float32),
        mesh=_MESH,
        compiler_params=pltpu.CompilerParams(use_tc_tiling_on_sc=False),
        scratch_types=[
            pltpu.VMEM((_NCH_FS, _K), jnp.int32),
            pltpu.VMEM((_NCH_FS, _K), jnp.int32),
            [pltpu.VMEM((_K, 64), jnp.float32)] * R,
            pltpu.VMEM_SHARED((_N, 64), jnp.float32),
            [pltpu.SemaphoreType.DMA] * R,
            [pltpu.SemaphoreType.DMA] * R,
        ],
    )
    def seg(table_hbm, gsrc_hbm, gdst_hbm, zeros_hbm, out_hbm,
            src_v, dst_v, rows, acc_sh, gsem, ssem):
        c = lax.axis_index("c")
        s = lax.axis_index("s")
        pltpu.sync_copy(zeros_hbm, acc_sh.at[pl.ds(s * _RSTRIDE, _RSPAN)])
        pltpu.sync_copy(gsrc_hbm.at[c, s], src_v)
        pltpu.sync_copy(gdst_hbm.at[c, s], dst_v)
        plsc.subcore_barrier()

        for b in range(R):
            pltpu.async_copy(table_hbm.at[src_v.at[b]], rows[b], gsem[b])

        def body(j, _):
            for b in range(R):
                i = j * R + b
                pltpu.make_async_copy(table_hbm.at[src_v.at[i]],
                                      rows[b], gsem[b]).wait()
                pltpu.async_copy(rows[b], acc_sh.at[dst_v.at[i]],
                                 ssem[b], add=True)
            for b in range(R):
                i = j * R + b
                pltpu.make_async_copy(rows[b], acc_sh.at[dst_v.at[i]],
                                      ssem[b]).wait()
                nxt = i + R

                @pl.when(nxt < _NCH_FS)
                def _():
                    pltpu.async_copy(table_hbm.at[src_v.at[nxt]],
                                     rows[b], gsem[b])
            return _

        lax.fori_loop(0, _NCH_FS // R, body, None)
        plsc.subcore_barrier()
        pltpu.sync_copy(acc_sh.at[pl.ds(s * _RSTRIDE, _RSPAN)],
                        out_hbm.at[c, pl.ds(s * _RSTRIDE, _RSPAN)])

    return seg


def _make_seg64x2():
    R = 5

    @functools.partial(
        pl.kernel,
        out_type=jax.ShapeDtypeStruct((2, _NC, _N, 64), jnp.float32),
        mesh=_MESH,
        compiler_params=pltpu.CompilerParams(use_tc_tiling_on_sc=False),
        scratch_types=[
            pltpu.VMEM((_NCH_FS, _K), jnp.int32),
            pltpu.VMEM((_NCH_FS, _K), jnp.int32),
            pltpu.VMEM((_NCH_FS, _K), jnp.int32),
            [pltpu.VMEM((_K, 64), jnp.float32)] * R,
            pltpu.VMEM_SHARED((_N, 64), jnp.float32),
            [pltpu.SemaphoreType.DMA] * R,
            [pltpu.SemaphoreType.DMA] * R,
        ],
    )
    def seg(table_hbm, gA_hbm, gB_hbm, gdst_hbm, zeros_hbm, out_hbm,
            srcA_v, srcB_v, dst_v, rows, acc_sh, gsem, ssem):
        c = lax.axis_index("c")
        s = lax.axis_index("s")
        pltpu.sync_copy(zeros_hbm, acc_sh.at[pl.ds(s * _RSTRIDE, _RSPAN)])
        pltpu.sync_copy(gA_hbm.at[c, s], srcA_v)
        pltpu.sync_copy(gB_hbm.at[c, s], srcB_v)
        pltpu.sync_copy(gdst_hbm.at[c, s], dst_v)
        plsc.subcore_barrier()

        def run_pass(src_v, acc_sh):
            for b in range(R):
                pltpu.async_copy(table_hbm.at[src_v.at[b]], rows[b], gsem[b])

            def body(j, _):
                for b in range(R):
                    i = j * R + b
                    pltpu.make_async_copy(table_hbm.at[src_v.at[i]],
                                          rows[b], gsem[b]).wait()
                    pltpu.async_copy(rows[b], acc_sh.at[dst_v.at[i]],
                                     ssem[b], add=True)
                for b in range(R):
                    i = j * R + b
                    pltpu.make_async_copy(rows[b], acc_sh.at[dst_v.at[i]],
                                          ssem[b]).wait()
                    nxt = i + R

                    @pl.when(nxt < _NCH_FS)
                    def _():
                        pltpu.async_copy(table_hbm.at[src_v.at[nxt]],
                                         rows[b], gsem[b])
                return _

            lax.fori_loop(0, _NCH_FS // R, body, None)

        run_pass(srcA_v, acc_sh)
        plsc.subcore_barrier()
        pltpu.sync_copy(acc_sh.at[pl.ds(s * _RSTRIDE, _RSPAN)],
                        out_hbm.at[0, c, pl.ds(s * _RSTRIDE, _RSPAN)])
        plsc.subcore_barrier()
        pltpu.sync_copy(zeros_hbm, acc_sh.at[pl.ds(s * _RSTRIDE, _RSPAN)])
        plsc.subcore_barrier()
        run_pass(srcB_v, acc_sh)
        plsc.subcore_barrier()
        pltpu.sync_copy(acc_sh.at[pl.ds(s * _RSTRIDE, _RSPAN)],
                        out_hbm.at[1, c, pl.ds(s * _RSTRIDE, _RSPAN)])

    return seg


def _make_seg_l45():
    R = 5
    CH = 40

    @functools.partial(
        pl.kernel,
        out_type=(jax.ShapeDtypeStruct((_NC, _N, 16), jnp.float32),
                  jax.ShapeDtypeStruct((_NC * _N, 16), jnp.float32)),
        mesh=_MESH,
        compiler_params=pltpu.CompilerParams(use_tc_tiling_on_sc=False),
        scratch_types=[
            pltpu.VMEM((_NCH_FS, _K), jnp.int32),
            pltpu.VMEM((_NCH_FS, _K), jnp.int32),
            pltpu.VMEM((_RSPAN,), jnp.int32),
            [pltpu.VMEM((_K, 64), jnp.float32)] * R,
            [pltpu.VMEM((_K, 16), jnp.float32)] * R,
            pltpu.VMEM_SHARED((_N, 64), jnp.float32),
            pltpu.VMEM_SHARED((_N, 16), jnp.float32),
            pltpu.VMEM((CH, 64), jnp.float32),
            pltpu.VMEM((CH, 16), jnp.float32),
            pltpu.VMEM((CH, 16), jnp.float32),
            pltpu.VMEM((CH, 16), jnp.float32),
            pltpu.VMEM((4, 16), jnp.float32),
            pltpu.VMEM((4, 16), jnp.float32),
            [pltpu.SemaphoreType.DMA] * R,
            [pltpu.SemaphoreType.DMA] * R,
        ],
    )
    def seg(table_hbm, gsrc_hbm, gdst_hbm, vidx_hbm, ri_hbm, ro_hbm,
            b4s_hbm, w5s_hbm, zeros_hbm, z16_hbm, out_hbm, v_hbm,
            src_v, dst_v, vidx_v, rows, rows16, acc_sh, acc16_sh,
            bufg, bufri, bufro, bufv, bufb4, bufw5, gsem, ssem):
        c = lax.axis_index("c")
        s = lax.axis_index("s")
        pltpu.sync_copy(zeros_hbm, acc_sh.at[pl.ds(s * _RSTRIDE, _RSPAN)])
        pltpu.sync_copy(z16_hbm, acc16_sh.at[pl.ds(s * _RSTRIDE, _RSPAN)])
        pltpu.sync_copy(gsrc_hbm.at[c, s], src_v)
        pltpu.sync_copy(gdst_hbm.at[c, s], dst_v)
        pltpu.sync_copy(vidx_hbm.at[c, s], vidx_v)
        pltpu.sync_copy(b4s_hbm.at[c], bufb4)
        pltpu.sync_copy(w5s_hbm.at[c], bufw5)
        plsc.subcore_barrier()

        for b in range(R):
            pltpu.async_copy(table_hbm.at[src_v.at[b]], rows[b], gsem[b])

        def body(j, _):
            for b in range(R):
                i = j * R + b
                pltpu.make_async_copy(table_hbm.at[src_v.at[i]],
                                      rows[b], gsem[b]).wait()
                pltpu.async_copy(rows[b], acc_sh.at[dst_v.at[i]],
                                 ssem[b], add=True)
            for b in range(R):
                i = j * R + b
                pltpu.make_async_copy(rows[b], acc_sh.at[dst_v.at[i]],
                                      ssem[b]).wait()
                nxt = i + R

                @pl.when(nxt < _NCH_FS)
                def _():
                    pltpu.async_copy(table_hbm.at[src_v.at[nxt]],
                                     rows[b], gsem[b])
            return _

        lax.fori_loop(0, _NCH_FS // R, body, None)
        plsc.subcore_barrier()

        b4v = [bufb4[j] for j in range(4)]
        w5v = [bufw5[j] for j in range(4)]

        for q in range(_RSPAN // CH):
            r0 = s * _RSTRIDE + q * CH
            pltpu.sync_copy(acc_sh.at[pl.ds(r0, CH)], bufg)
            pltpu.sync_copy(ri_hbm.at[pl.ds(r0, CH)], bufri)
            pltpu.sync_copy(ro_hbm.at[pl.ds(r0, CH)], bufro)

            def row(i, _):
                riv = bufri[i]
                rov = bufro[i]
                acc = None
                for j in range(4):
                    g = bufg[i, pl.ds(16 * j, 16)]
                    h = jnp.maximum(g * riv + b4v[j], 0.0) * rov
                    t = h * w5v[j]
                    acc = t if acc is None else acc + t
                bufv[i] = acc
                return _

            lax.fori_loop(0, CH, row, None)
            pltpu.sync_copy(bufv, v_hbm.at[vidx_v.at[pl.ds(q * CH, CH)]])
        plsc.subcore_barrier()

        for b in range(R):
            pltpu.async_copy(v_hbm.at[src_v.at[b]], rows16[b], gsem[b])

        def body3(j, _):
            for b in range(R):
                i = j * R + b
                pltpu.make_async_copy(v_hbm.at[src_v.at[i]],
                                      rows16[b], gsem[b]).wait()
                pltpu.async_copy(rows16[b], acc16_sh.at[dst_v.at[i]],
                                 ssem[b], add=True)
            for b in range(R):
                i = j * R + b
                pltpu.make_async_copy(rows16[b], acc16_sh.at[dst_v.at[i]],
                                      ssem[b]).wait()
                nxt = i + R

                @pl.when(nxt < _NCH_FS)
                def _():
                    pltpu.async_copy(v_hbm.at[src_v.at[nxt]],
                                     rows16[b], gsem[b])
            return _

        lax.fori_loop(0, _NCH_FS // R, body3, None)
        plsc.subcore_barrier()
        pltpu.sync_copy(acc16_sh.at[pl.ds(s * _RSTRIDE, _RSPAN)],
                        out_hbm.at[c, pl.ds(s * _RSTRIDE, _RSPAN)])

    return seg


def _make_deg():
    @functools.partial(
        pl.kernel,
        out_type=jax.ShapeDtypeStruct((_NC, _N, 16), jnp.float32),
        mesh=_MESH,
        compiler_params=pltpu.CompilerParams(use_tc_tiling_on_sc=False),
        scratch_types=[
            pltpu.VMEM((_NCH_ES, _K), jnp.int32),
            pltpu.VMEM((_NCH_ES, _K), jnp.int32),
            pltpu.VMEM((_K, 16), jnp.float32),
            pltpu.VMEM((_K, 16), jnp.float32),
            pltpu.VMEM_SHARED((_N, 16), jnp.float32),
            [pltpu.SemaphoreType.DMA] * 5,
            [pltpu.SemaphoreType.DMA] * 5,
        ],
    )
    def deg(gsrc_hbm, gdst_hbm, oneslr_hbm, zeros_hbm, d_hbm,
            src_v, dst_v, rows_s, rows_d, acc_sh, ssem, dsem):
        R = 5
        c = lax.axis_index("c")
        s = lax.axis_index("s")
        pltpu.sync_copy(zeros_hbm, acc_sh.at[pl.ds(s * _RSTRIDE, _RSPAN)])
        pltpu.sync_copy(oneslr_hbm.at[0], rows_s)
        pltpu.sync_copy(oneslr_hbm.at[1], rows_d)
        pltpu.sync_copy(gsrc_hbm.at[c, s], src_v)
        pltpu.sync_copy(gdst_hbm.at[c, s], dst_v)
        plsc.subcore_barrier()

        for b in range(R):
            pltpu.async_copy(rows_s, acc_sh.at[src_v.at[b]], ssem[b],
                             add=True)
            pltpu.async_copy(rows_d, acc_sh.at[dst_v.at[b]], dsem[b],
                             add=True)

        def body(j, _):
            for b in range(R):
                i = j * R + b
                pltpu.make_async_copy(rows_s, acc_sh.at[src_v.at[i]],
                                      ssem[b]).wait()
                pltpu.make_async_copy(rows_d, acc_sh.at[dst_v.at[i]],
                                      dsem[b]).wait()
                nxt = i + R

                @pl.when(nxt < _NCH_ES)
                def _():
                    pltpu.async_copy(rows_s, acc_sh.at[src_v.at[nxt]],
                                     ssem[b], add=True)
                    pltpu.async_copy(rows_d, acc_sh.at[dst_v.at[nxt]],
                                     dsem[b], add=True)
            return _

        lax.fori_loop(0, _NCH_ES // 5, body, None)
        plsc.subcore_barrier()
        pltpu.sync_copy(acc_sh.at[pl.ds(s * _RSTRIDE, _RSPAN)],
                        d_hbm.at[c, pl.ds(s * _RSTRIDE, _RSPAN)])

    return deg


_BM = 1000
_G = _N // _BM


def _rb(d):
    return pl.BlockSpec((_BM, d), lambda i: (i, 0))


def _full(shape):
    nd = len(shape)
    return pl.BlockSpec(shape, lambda i: (0,) * nd)


def _prep_scale(d, in_feat):
    def body(d0, d1, x_ref, ro_ref, ri_ref, t1_ref):
        dsum = d0[...] + d1[...]
        odeg = jnp.maximum(dsum[:, 0:1], 1.0)
        ideg = jnp.maximum(dsum[:, 8:9], 1.0)
        ro = lax.rsqrt(odeg)
        ro_ref[...] = jnp.broadcast_to(ro, ro_ref.shape)
        ri_ref[...] = jnp.broadcast_to(lax.rsqrt(ideg), ri_ref.shape)
        t1_ref[...] = x_ref[...] * ro

    return pl.pallas_call(
        body,
        grid=(_G,),
        in_specs=[_rb(16)] * 2 + [_rb(128)],
        out_specs=[_rb(16), _rb(16), _rb(128)],
        out_shape=[jax.ShapeDtypeStruct((_N, 16), jnp.float32)] * 2
        + [jax.ShapeDtypeStruct((_N, 128), jnp.float32)],
    )(d[0], d[1], in_feat)


def _stage_mm(agg, ri, ro, W, b):
    do = W.shape[1]

    def body(a0, a1, ri_ref, ro_ref, w_ref, b_ref, o_ref):
        x0 = a0[...] * ri_ref[:, :1]
        x1 = a1[...] * ri_ref[:, :1]
        y = (jnp.dot(x0, w_ref[0:64, :], preferred_element_type=jnp.float32)
             + jnp.dot(x1, w_ref[64:128, :],
                       preferred_element_type=jnp.float32))
        o_ref[...] = jax.nn.relu(y + b_ref[...]) * ro_ref[:, :1]

    return pl.pallas_call(
        body,
        grid=(_G,),
        in_specs=[_rb(64), _rb(64), _rb(16), _rb(16),
                  _full((128, do)), _full((1, do))],
        out_specs=_rb(do),
        out_shape=jax.ShapeDtypeStruct((_N, do), jnp.float32),
    )(agg[0], agg[1], ri, ro, W, b[None, :])


def _stage_l3l4(aggA, aggB, ri, ro, W3, b3, W4):
    def body(a0, a1, a2, a3, ri_ref, ro_ref, w3_ref, b3_ref, w4_ref, o_ref):
        t = None
        for q, aq in enumerate((a0, a1, a2, a3)):
            xq = aq[...] * ri_ref[:, :1]
            yq = jnp.dot(xq, w3_ref[64 * q:64 * (q + 1), :],
                         preferred_element_type=jnp.float32)
            t = yq if t is None else t + yq
        t = jax.nn.relu(t + b3_ref[...]) * ro_ref[:, :1]
        o_ref[...] = jnp.dot(t, w4_ref[...],
                             preferred_element_type=jnp.float32)

    return pl.pallas_call(
        body,
        grid=(_G,),
        in_specs=[_rb(64)] * 4 + [_rb(16), _rb(16),
                                  _full((256, 256)), _full((1, 256)),
                                  _full((256, 128))],
        out_specs=_rb(128),
        out_shape=jax.ShapeDtypeStruct((_N, 128), jnp.float32),
    )(aggA[0], aggA[1], aggB[0], aggB[1], ri, ro, W3, b3[None, :], W4)


def _stage_out(agg, ri, b5):
    def body(a0, a1, ri_ref, b5_ref, o_ref):
        t = jnp.sum(a0[...] + a1[...], axis=1, keepdims=True)
        v = jax.nn.relu(t * ri_ref[:, :1] + b5_ref[0, 0])
        o_ref[...] = jnp.broadcast_to(v, o_ref.shape)

    return pl.pallas_call(
        body,
        grid=(_G,),
        in_specs=[_rb(16), _rb(16), _rb(16), _full((1, 1))],
        out_specs=_rb(16),
        out_shape=jax.ShapeDtypeStruct((_N, 16), jnp.float32),
    )(agg[0], agg[1], ri, b5[None, :])


def kernel(in_feat, edge_index, W1, b1, W2, b2, W3, b3, W4, b4, W5, b5):
    ei = edge_index.astype(jnp.int32)
    src, dst = ei[0], ei[1]

    def fs(a, b):
        return jnp.stack([a, b]).reshape(_NC, _NS, _NCH_FS, _K)

    g_h = fs(2 * src, 2 * src + 1)
    gA = fs(4 * src, 4 * src + 1)
    gB = fs(4 * src + 2, 4 * src + 3)
    gd_fs = fs(dst, dst)
    src_es = src.reshape(_NC, _NS, _NCH_ES, _K)
    dst_es = dst.reshape(_NC, _NS, _NCH_ES, _K)

    z64 = jnp.zeros((_RSPAN, 64), jnp.float32)
    z16 = jnp.zeros((_RSPAN, 16), jnp.float32)
    lane = jnp.arange(16)
    oneslr = jnp.stack([
        jnp.broadcast_to((lane < 8).astype(jnp.float32), (_K, 16)),
        jnp.broadcast_to((lane >= 8).astype(jnp.float32), (_K, 16)),
    ])

    seg64 = _make_seg64()
    seg64x2 = _make_seg64x2()
    segl45 = _make_seg_l45()
    deg = _make_deg()

    d = deg(src_es, dst_es, oneslr, z16)
    ro, ri, t1 = _prep_scale(d, in_feat)
    a1 = seg64(t1.reshape(_NC * _N, 64), g_h, gd_fs, z64)
    t2 = _stage_mm(a1, ri, ro, W1, b1)
    a2 = seg64(t2.reshape(_NC * _N, 64), g_h, gd_fs, z64)
    t3 = _stage_mm(a2, ri, ro, W2, b2)
    t3q = t3.reshape(4 * _N, 64)
    a3 = seg64x2(t3q, gA, gB, gd_fs, z64)
    u = _stage_l3l4(a3[0], a3[1], ri, ro, W3, b3, W4)
    rr = (jnp.arange(_NS, dtype=jnp.int32)[:, None] * _RSTRIDE
          + jnp.arange(_RSPAN, dtype=jnp.int32)[None, :])
    vidx = jnp.stack([2 * rr, 2 * rr + 1])
    b4s = b4.reshape(_NC, 4, 16)
    w5s = W5[:, 0].reshape(_NC, 4, 16)
    a5, _unused_v = segl45(u.reshape(_NC * _N, 64), g_h, gd_fs, vidx,
                           ri, ro, b4s, w5s, z64, z16)
    out16 = _stage_out(a5, ri, b5)
    return out16[:, :1]


# speedup vs baseline: 1.0380x; 1.0080x over previous
import functools
import jax
import jax.numpy as jnp
from jax import lax
from jax.experimental import pallas as pl
from jax.experimental.pallas import tpu as pltpu
from jax.experimental.pallas import tpu_sc as plsc

_N = 10000
_E = 320000
_NC = 2
_NS = 16
_K = 80
_RSTRIDE = 624
_RSPAN = 640

_NCH_FS = _E // (_NS * _K)
_NCH_ES = _E // (_NC * _NS * _K)

_MESH = plsc.VectorSubcoreMesh(core_axis_name="c", subcore_axis_name="s")


def _make_seg64():
    R = 5

    @functools.partial(
        pl.kernel,
        out_type=jax.ShapeDtypeStruct((_NC, _N, 64), jnp.float32),
        mesh=_MESH,
        compiler_params=pltpu.CompilerParams(use_tc_tiling_on_sc=False),
        scratch_types=[
            pltpu.VMEM((_NCH_FS, _K), jnp.int32),
            pltpu.VMEM((_NCH_FS, _K), jnp.int32),
            [pltpu.VMEM((_K, 64), jnp.float32)] * R,
            pltpu.VMEM_SHARED((_N, 64), jnp.float32),
            [pltpu.SemaphoreType.DMA] * R,
            [pltpu.SemaphoreType.DMA] * R,
        ],
    )
    def seg(table_hbm, gsrc_hbm, gdst_hbm, zeros_hbm, out_hbm,
            src_v, dst_v, rows, acc_sh, gsem, ssem):
        c = lax.axis_index("c")
        s = lax.axis_index("s")
        pltpu.sync_copy(zeros_hbm, acc_sh.at[pl.ds(s * _RSTRIDE, _RSPAN)])
        pltpu.sync_copy(gsrc_hbm.at[c, s], src_v)
        pltpu.sync_copy(gdst_hbm.at[c, s], dst_v)
        plsc.subcore_barrier()

        for b in range(R):
            pltpu.async_copy(table_hbm.at[src_v.at[b]], rows[b], gsem[b])

        def body(j, _):
            for b in range(R):
                i = j * R + b
                pltpu.make_async_copy(table_hbm.at[src_v.at[i]],
                                      rows[b], gsem[b]).wait()
                pltpu.async_copy(rows[b], acc_sh.at[dst_v.at[i]],
                                 ssem[b], add=True)
            for b in range(R):
                i = j * R + b
                pltpu.make_async_copy(rows[b], acc_sh.at[dst_v.at[i]],
                                      ssem[b]).wait()
                nxt = i + R

                @pl.when(nxt < _NCH_FS)
                def _():
                    pltpu.async_copy(table_hbm.at[src_v.at[nxt]],
                                     rows[b], gsem[b])
            return _

        lax.fori_loop(0, _NCH_FS // R, body, None)
        plsc.subcore_barrier()
        pltpu.sync_copy(acc_sh.at[pl.ds(s * _RSTRIDE, _RSPAN)],
                        out_hbm.at[c, pl.ds(s * _RSTRIDE, _RSPAN)])

    return seg


def _make_seg64x2():
    R = 5

    @functools.partial(
        pl.kernel,
        out_type=jax.ShapeDtypeStruct((2, _NC, _N, 64), jnp.float32),
        mesh=_MESH,
        compiler_params=pltpu.CompilerParams(use_tc_tiling_on_sc=False),
        scratch_types=[
            pltpu.VMEM((_NCH_FS, _K), jnp.int32),
            pltpu.VMEM((_NCH_FS, _K), jnp.int32),
            pltpu.VMEM((_NCH_FS, _K), jnp.int32),
            [pltpu.VMEM((_K, 64), jnp.float32)] * R,
            pltpu.VMEM_SHARED((_N, 64), jnp.float32),
            [pltpu.SemaphoreType.DMA] * R,
            [pltpu.SemaphoreType.DMA] * R,
        ],
    )
    def seg(table_hbm, gA_hbm, gB_hbm, gdst_hbm, zeros_hbm, out_hbm,
            srcA_v, srcB_v, dst_v, rows, acc_sh, gsem, ssem):
        c = lax.axis_index("c")
        s = lax.axis_index("s")
        pltpu.sync_copy(zeros_hbm, acc_sh.at[pl.ds(s * _RSTRIDE, _RSPAN)])
        pltpu.sync_copy(gA_hbm.at[c, s], srcA_v)
        pltpu.sync_copy(gB_hbm.at[c, s], srcB_v)
        pltpu.sync_copy(gdst_hbm.at[c, s], dst_v)
        plsc.subcore_barrier()

        def run_pass(src_v, acc_sh):
            for b in range(R):
                pltpu.async_copy(table_hbm.at[src_v.at[b]], rows[b], gsem[b])

            def body(j, _):
                for b in range(R):
                    i = j * R + b
                    pltpu.make_async_copy(table_hbm.at[src_v.at[i]],
                                          rows[b], gsem[b]).wait()
                    pltpu.async_copy(rows[b], acc_sh.at[dst_v.at[i]],
                                     ssem[b], add=True)
                for b in range(R):
                    i = j * R + b
                    pltpu.make_async_copy(rows[b], acc_sh.at[dst_v.at[i]],
                                          ssem[b]).wait()
                    nxt = i + R

                    @pl.when(nxt < _NCH_FS)
                    def _():
                        pltpu.async_copy(table_hbm.at[src_v.at[nxt]],
                                         rows[b], gsem[b])
                return _

            lax.fori_loop(0, _NCH_FS // R, body, None)

        run_pass(srcA_v, acc_sh)
        plsc.subcore_barrier()
        pltpu.sync_copy(acc_sh.at[pl.ds(s * _RSTRIDE, _RSPAN)],
                        out_hbm.at[0, c, pl.ds(s * _RSTRIDE, _RSPAN)])
        plsc.subcore_barrier()
        pltpu.sync_copy(zeros_hbm, acc_sh.at[pl.ds(s * _RSTRIDE, _RSPAN)])
        plsc.subcore_barrier()
        run_pass(srcB_v, acc_sh)
        plsc.subcore_barrier()
        pltpu.sync_copy(acc_sh.at[pl.ds(s * _RSTRIDE, _RSPAN)],
                        out_hbm.at[1, c, pl.ds(s * _RSTRIDE, _RSPAN)])

    return seg


def _make_seg_l45():
    R = 5
    CH = 64

    @functools.partial(
        pl.kernel,
        out_type=(jax.ShapeDtypeStruct((_NC, _N, 16), jnp.float32),
                  jax.ShapeDtypeStruct((_NC * _N, 16), jnp.float32)),
        mesh=_MESH,
        compiler_params=pltpu.CompilerParams(use_tc_tiling_on_sc=False),
        scratch_types=[
            pltpu.VMEM((_NCH_FS, _K), jnp.int32),
            pltpu.VMEM((_NCH_FS, _K), jnp.int32),
            pltpu.VMEM((_RSPAN,), jnp.int32),
            [pltpu.VMEM((_K, 64), jnp.float32)] * R,
            [pltpu.VMEM((_K, 16), jnp.float32)] * R,
            pltpu.VMEM_SHARED((_N, 64), jnp.float32),
            pltpu.VMEM_SHARED((_N, 16), jnp.float32),
            pltpu.VMEM((CH, 64), jnp.float32),
            pltpu.VMEM((CH, 16), jnp.float32),
            pltpu.VMEM((CH, 16), jnp.float32),
            pltpu.VMEM((CH, 16), jnp.float32),
            pltpu.VMEM((4, 16), jnp.float32),
            pltpu.VMEM((4, 16), jnp.float32),
            [pltpu.SemaphoreType.DMA] * R,
            [pltpu.SemaphoreType.DMA] * R,
        ],
    )
    def seg(table_hbm, gsrc_hbm, gdst_hbm, vidx_hbm, ri_hbm, ro_hbm,
            b4s_hbm, w5s_hbm, zeros_hbm, z16_hbm, out_hbm, v_hbm,
            src_v, dst_v, vidx_v, rows, rows16, acc_sh, acc16_sh,
            bufg, bufri, bufro, bufv, bufb4, bufw5, gsem, ssem):
        c = lax.axis_index("c")
        s = lax.axis_index("s")
        pltpu.sync_copy(zeros_hbm, acc_sh.at[pl.ds(s * _RSTRIDE, _RSPAN)])
        pltpu.sync_copy(z16_hbm, acc16_sh.at[pl.ds(s * _RSTRIDE, _RSPAN)])
        pltpu.sync_copy(gsrc_hbm.at[c, s], src_v)
        pltpu.sync_copy(gdst_hbm.at[c, s], dst_v)
        pltpu.sync_copy(vidx_hbm.at[c, s], vidx_v)
        pltpu.sync_copy(b4s_hbm.at[c], bufb4)
        pltpu.sync_copy(w5s_hbm.at[c], bufw5)
        plsc.subcore_barrier()

        for b in range(R):
            pltpu.async_copy(table_hbm.at[src_v.at[b]], rows[b], gsem[b])

        def body(j, _):
            for b in range(R):
                i = j * R + b
                pltpu.make_async_copy(table_hbm.at[src_v.at[i]],
                                      rows[b], gsem[b]).wait()
                pltpu.async_copy(rows[b], acc_sh.at[dst_v.at[i]],
                                 ssem[b], add=True)
            for b in range(R):
                i = j * R + b
                pltpu.make_async_copy(rows[b], acc_sh.at[dst_v.at[i]],
                                      ssem[b]).wait()
                nxt = i + R

                @pl.when(nxt < _NCH_FS)
                def _():
                    pltpu.async_copy(table_hbm.at[src_v.at[nxt]],
                                     rows[b], gsem[b])
            return _

        lax.fori_loop(0, _NCH_FS // R, body, None)
        plsc.subcore_barrier()

        b4v = [bufb4[j] for j in range(4)]
        w5v = [bufw5[j] for j in range(4)]

        for q in range(_RSPAN // CH):
            r0 = s * _RSTRIDE + q * CH
            pltpu.sync_copy(acc_sh.at[pl.ds(r0, CH)], bufg)
            pltpu.sync_copy(ri_hbm.at[pl.ds(r0, CH)], bufri)
            pltpu.sync_copy(ro_hbm.at[pl.ds(r0, CH)], bufro)

            def row(i, _):
                riv = bufri[i]
                rov = bufro[i]
                acc = None
                for j in range(4):
                    g = bufg[i, pl.ds(16 * j, 16)]
                    h = jnp.maximum(g * riv + b4v[j], 0.0) * rov
                    t = h * w5v[j]
                    acc = t if acc is None else acc + t
                bufv[i] = acc
                return _

            lax.fori_loop(0, CH, row, None)
            pltpu.sync_copy(bufv, v_hbm.at[vidx_v.at[pl.ds(q * CH, CH)]])
        plsc.subcore_barrier()

        for b in range(R):
            pltpu.async_copy(v_hbm.at[src_v.at[b]], rows16[b], gsem[b])

        def body3(j, _):
            for b in range(R):
                i = j * R + b
                pltpu.make_async_copy(v_hbm.at[src_v.at[i]],
                                      rows16[b], gsem[b]).wait()
                pltpu.async_copy(rows16[b], acc16_sh.at[dst_v.at[i]],
                                 ssem[b], add=True)
            for b in range(R):
                i = j * R + b
                pltpu.make_async_copy(rows16[b], acc16_sh.at[dst_v.at[i]],
                                      ssem[b]).wait()
                nxt = i + R

                @pl.when(nxt < _NCH_FS)
                def _():
                    pltpu.async_copy(v_hbm.at[src_v.at[nxt]],
                                     rows16[b], gsem[b])
            return _

        lax.fori_loop(0, _NCH_FS // R, body3, None)
        plsc.subcore_barrier()
        pltpu.sync_copy(acc16_sh.at[pl.ds(s * _RSTRIDE, _RSPAN)],
                        out_hbm.at[c, pl.ds(s * _RSTRIDE, _RSPAN)])

    return seg


def _make_deg():
    @functools.partial(
        pl.kernel,
        out_type=jax.ShapeDtypeStruct((_NC, _N, 16), jnp.float32),
        mesh=_MESH,
        compiler_params=pltpu.CompilerParams(use_tc_tiling_on_sc=False),
        scratch_types=[
            pltpu.VMEM((_NCH_ES, _K), jnp.int32),
            pltpu.VMEM((_NCH_ES, _K), jnp.int32),
            pltpu.VMEM((_K, 16), jnp.float32),
            pltpu.VMEM((_K, 16), jnp.float32),
            pltpu.VMEM_SHARED((_N, 16), jnp.float32),
            [pltpu.SemaphoreType.DMA] * 5,
            [pltpu.SemaphoreType.DMA] * 5,
        ],
    )
    def deg(gsrc_hbm, gdst_hbm, oneslr_hbm, zeros_hbm, d_hbm,
            src_v, dst_v, rows_s, rows_d, acc_sh, ssem, dsem):
        R = 5
        c = lax.axis_index("c")
        s = lax.axis_index("s")
        pltpu.sync_copy(zeros_hbm, acc_sh.at[pl.ds(s * _RSTRIDE, _RSPAN)])
        pltpu.sync_copy(oneslr_hbm.at[0], rows_s)
        pltpu.sync_copy(oneslr_hbm.at[1], rows_d)
        pltpu.sync_copy(gsrc_hbm.at[c, s], src_v)
        pltpu.sync_copy(gdst_hbm.at[c, s], dst_v)
        plsc.subcore_barrier()

        for b in range(R):
            pltpu.async_copy(rows_s, acc_sh.at[src_v.at[b]], ssem[b],
                             add=True)
            pltpu.async_copy(rows_d, acc_sh.at[dst_v.at[b]], dsem[b],
                             add=True)

        def body(j, _):
            for b in range(R):
                i = j * R + b
                pltpu.make_async_copy(rows_s, acc_sh.at[src_v.at[i]],
                                      ssem[b]).wait()
                pltpu.make_async_copy(rows_d, acc_sh.at[dst_v.at[i]],
                                      dsem[b]).wait()
                nxt = i + R

                @pl.when(nxt < _NCH_ES)
                def _():
                    pltpu.async_copy(rows_s, acc_sh.at[src_v.at[nxt]],
                                     ssem[b], add=True)
                    pltpu.async_copy(rows_d, acc_sh.at[dst_v.at[nxt]],
                                     dsem[b], add=True)
            return _

        lax.fori_loop(0, _NCH_ES // 5, body, None)
        plsc.subcore_barrier()
        pltpu.sync_copy(acc_sh.at[pl.ds(s * _RSTRIDE, _RSPAN)],
                        d_hbm.at[c, pl.ds(s * _RSTRIDE, _RSPAN)])

    return deg


_BM = 1000
_G = _N // _BM


def _rb(d):
    return pl.BlockSpec((_BM, d), lambda i: (i, 0))


def _full(shape):
    nd = len(shape)
    return pl.BlockSpec(shape, lambda i: (0,) * nd)


def _prep_scale(d, in_feat):
    def body(d0, d1, x_ref, ro_ref, ri_ref, t1_ref):
        dsum = d0[...] + d1[...]
        odeg = jnp.maximum(dsum[:, 0:1], 1.0)
        ideg = jnp.maximum(dsum[:, 8:9], 1.0)
        ro = lax.rsqrt(odeg)
        ro_ref[...] = jnp.broadcast_to(ro, ro_ref.shape)
        ri_ref[...] = jnp.broadcast_to(lax.rsqrt(ideg), ri_ref.shape)
        t1_ref[...] = x_ref[...] * ro

    return pl.pallas_call(
        body,
        grid=(_G,),
        in_specs=[_rb(16)] * 2 + [_rb(128)],
        out_specs=[_rb(16), _rb(16), _rb(128)],
        out_shape=[jax.ShapeDtypeStruct((_N, 16), jnp.float32)] * 2
        + [jax.ShapeDtypeStruct((_N, 128), jnp.float32)],
    )(d[0], d[1], in_feat)


def _stage_mm(agg, ri, ro, W, b):
    do = W.shape[1]

    def body(a0, a1, ri_ref, ro_ref, w_ref, b_ref, o_ref):
        x0 = a0[...] * ri_ref[:, :1]
        x1 = a1[...] * ri_ref[:, :1]
        y = (jnp.dot(x0, w_ref[0:64, :], preferred_element_type=jnp.float32)
             + jnp.dot(x1, w_ref[64:128, :],
                       preferred_element_type=jnp.float32))
        o_ref[...] = jax.nn.relu(y + b_ref[...]) * ro_ref[:, :1]

    return pl.pallas_call(
        body,
        grid=(_G,),
        in_specs=[_rb(64), _rb(64), _rb(16), _rb(16),
                  _full((128, do)), _full((1, do))],
        out_specs=_rb(do),
        out_shape=jax.ShapeDtypeStruct((_N, do), jnp.float32),
    )(agg[0], agg[1], ri, ro, W, b[None, :])


def _stage_l3l4(aggA, aggB, ri, ro, W3, b3, W4):
    def body(a0, a1, a2, a3, ri_ref, ro_ref, w3_ref, b3_ref, w4_ref, o_ref):
        t = None
        for q, aq in enumerate((a0, a1, a2, a3)):
            xq = aq[...] * ri_ref[:, :1]
            yq = jnp.dot(xq, w3_ref[64 * q:64 * (q + 1), :],
                         preferred_element_type=jnp.float32)
            t = yq if t is None else t + yq
        t = jax.nn.relu(t + b3_ref[...]) * ro_ref[:, :1]
        o_ref[...] = jnp.dot(t, w4_ref[...],
                             preferred_element_type=jnp.float32)

    return pl.pallas_call(
        body,
        grid=(_G,),
        in_specs=[_rb(64)] * 4 + [_rb(16), _rb(16),
                                  _full((256, 256)), _full((1, 256)),
                                  _full((256, 128))],
        out_specs=_rb(128),
        out_shape=jax.ShapeDtypeStruct((_N, 128), jnp.float32),
    )(aggA[0], aggA[1], aggB[0], aggB[1], ri, ro, W3, b3[None, :], W4)


def _stage_out(agg, ri, b5):
    def body(a0, a1, ri_ref, b5_ref, o_ref):
        t = jnp.sum(a0[...] + a1[...], axis=1, keepdims=True)
        v = jax.nn.relu(t * ri_ref[:, :1] + b5_ref[0, 0])
        o_ref[...] = jnp.broadcast_to(v, o_ref.shape)

    return pl.pallas_call(
        body,
        grid=(_G,),
        in_specs=[_rb(16), _rb(16), _rb(16), _full((1, 1))],
        out_specs=_rb(16),
        out_shape=jax.ShapeDtypeStruct((_N, 16), jnp.float32),
    )(agg[0], agg[1], ri, b5[None, :])


def kernel(in_feat, edge_index, W1, b1, W2, b2, W3, b3, W4, b4, W5, b5):
    ei = edge_index.astype(jnp.int32)
    src, dst = ei[0], ei[1]

    def fs(a, b):
        return jnp.stack([a, b]).reshape(_NC, _NS, _NCH_FS, _K)

    g_h = fs(2 * src, 2 * src + 1)
    gA = fs(4 * src, 4 * src + 1)
    gB = fs(4 * src + 2, 4 * src + 3)
    gd_fs = fs(dst, dst)
    src_es = src.reshape(_NC, _NS, _NCH_ES, _K)
    dst_es = dst.reshape(_NC, _NS, _NCH_ES, _K)

    z64 = jnp.zeros((_RSPAN, 64), jnp.float32)
    z16 = jnp.zeros((_RSPAN, 16), jnp.float32)
    lane = jnp.arange(16)
    oneslr = jnp.stack([
        jnp.broadcast_to((lane < 8).astype(jnp.float32), (_K, 16)),
        jnp.broadcast_to((lane >= 8).astype(jnp.float32), (_K, 16)),
    ])

    seg64 = _make_seg64()
    seg64x2 = _make_seg64x2()
    segl45 = _make_seg_l45()
    deg = _make_deg()

    d = deg(src_es, dst_es, oneslr, z16)
    ro, ri, t1 = _prep_scale(d, in_feat)
    a1 = seg64(t1.reshape(_NC * _N, 64), g_h, gd_fs, z64)
    t2 = _stage_mm(a1, ri, ro, W1, b1)
    a2 = seg64(t2.reshape(_NC * _N, 64), g_h, gd_fs, z64)
    t3 = _stage_mm(a2, ri, ro, W2, b2)
    t3q = t3.reshape(4 * _N, 64)
    a3 = seg64x2(t3q, gA, gB, gd_fs, z64)
    u = _stage_l3l4(a3[0], a3[1], ri, ro, W3, b3, W4)
    rr = (jnp.arange(_NS, dtype=jnp.int32)[:, None] * _RSTRIDE
          + jnp.arange(_RSPAN, dtype=jnp.int32)[None, :])
    vidx = jnp.stack([2 * rr, 2 * rr + 1])
    b4s = b4.reshape(_NC, 4, 16)
    w5s = W5[:, 0].reshape(_NC, 4, 16)
    a5, _unused_v = segl45(u.reshape(_NC * _N, 64), g_h, gd_fs, vidx,
                           ri, ro, b4s, w5s, z64, z16)
    out16 = _stage_out(a5, ri, b5)
    return out16[:, :1]
